# Initial kernel scaffold; baseline (speedup 1.0000x reference)
import jax
import jax.numpy as jnp
from jax.experimental import pallas as pl


def kernel(positions, weight):
    raise NotImplementedError("write your pallas kernel here")


# speedup vs baseline: 1.9878x; 1.9878x over previous
import functools

import jax
import jax.numpy as jnp
from jax import lax
from jax.experimental import pallas as pl
from jax.experimental.pallas import tpu as pltpu
from jax.experimental.pallas import tpu_sc as plsc

_INFO = plsc.get_sparse_core_info()
_NC = _INFO.num_cores
_NS = _INFO.num_subcores
_NW = _NC * _NS


def _gather_call(positions_flat, weight, chunk):
    b_total = positions_flat.shape[0]
    d = weight.shape[1]
    b_per_w = b_total // _NW
    nchunk = b_per_w // chunk
    pos3 = positions_flat.reshape(_NW, nchunk, chunk)
    mesh = plsc.VectorSubcoreMesh(core_axis_name="c", subcore_axis_name="s")

    @functools.partial(
        pl.kernel,
        mesh=mesh,
        out_type=jax.ShapeDtypeStruct((b_total, d), jnp.float32),
        scratch_types=[
            pltpu.VMEM((nchunk, chunk), jnp.int32),
            pltpu.VMEM((chunk, d), jnp.float32),
            pltpu.SemaphoreType.DMA,
        ],
    )
    def gather_kernel(pos_hbm, table_hbm, out_hbm, idx_v, buf, gsem):
        wid = lax.axis_index("s") * _NC + lax.axis_index("c")
        base = wid * b_per_w
        pltpu.sync_copy(pos_hbm.at[wid], idx_v)

        def body(c, carry):
            pltpu.async_copy(table_hbm.at[idx_v.at[c]], buf, gsem).wait()
            pltpu.sync_copy(buf, out_hbm.at[pl.ds(base + c * chunk, chunk)])
            return carry

        lax.fori_loop(0, nchunk, body, 0)

    return gather_kernel(pos3, weight)


def kernel(positions, weight):
    flat = positions.reshape(-1)
    out = _gather_call(flat, weight, chunk=32)
    return out.reshape(positions.shape + (weight.shape[1],))


# speedup vs baseline: 2.3818x; 1.1982x over previous
import functools

import jax
import jax.numpy as jnp
from jax import lax
from jax.experimental import pallas as pl
from jax.experimental.pallas import tpu as pltpu
from jax.experimental.pallas import tpu_sc as plsc

_INFO = plsc.get_sparse_core_info()
_NC = _INFO.num_cores
_NS = _INFO.num_subcores
_NW = _NC * _NS

_NB = 4


def _gather_call(positions_flat, weight, chunk):
    b_total = positions_flat.shape[0]
    d = weight.shape[1]
    b_per_w = b_total // _NW
    nchunk = b_per_w // chunk
    nouter = nchunk // _NB
    pos3 = positions_flat.reshape(_NW, nchunk, chunk)
    mesh = plsc.VectorSubcoreMesh(core_axis_name="c", subcore_axis_name="s")

    scratch = (
        [pltpu.VMEM((nchunk, chunk), jnp.int32)]
        + [pltpu.VMEM((chunk, d), jnp.float32) for _ in range(_NB)]
        + [pltpu.SemaphoreType.DMA for _ in range(2 * _NB)]
    )

    @functools.partial(
        pl.kernel,
        mesh=mesh,
        out_type=jax.ShapeDtypeStruct((b_total, d), jnp.float32),
        scratch_types=scratch,
    )
    def gather_kernel(pos_hbm, table_hbm, out_hbm, idx_v, *rest):
        bufs = rest[:_NB]
        gsems = rest[_NB:2 * _NB]
        wsems = rest[2 * _NB:]

        wid = lax.axis_index("s") * _NC + lax.axis_index("c")
        base = wid * b_per_w
        pltpu.sync_copy(pos_hbm.at[wid], idx_v)

        def start_g(c, b):
            pltpu.async_copy(table_hbm.at[idx_v.at[c]], bufs[b], gsems[b])

        def wait_g(c, b):
            pltpu.make_async_copy(table_hbm.at[idx_v.at[c]], bufs[b],
                                  gsems[b]).wait()

        def start_w(c, b):
            pltpu.async_copy(bufs[b], out_hbm.at[pl.ds(base + c * chunk, chunk)],
                             wsems[b])

        def wait_w(b):
            pltpu.make_async_copy(bufs[b], out_hbm.at[pl.ds(base, chunk)],
                                  wsems[b]).wait()

        for b in range(_NB):
            start_g(b, b)

        def body(o, carry):
            for b in range(_NB):
                c = o * _NB + b
                if b == 0:
                    @pl.when(o > 0)
                    def _():
                        wait_w(_NB - 1)
                        start_g(c - 1 + _NB, _NB - 1)
                else:
                    @pl.when(o < nouter - 1)
                    def _():
                        wait_w(b - 1)
                        start_g(c - 1 + _NB, b - 1)
                wait_g(c, b)
                start_w(c, b)
            return carry

        lax.fori_loop(0, nouter, body, 0)

        for b in range(_NB):
            wait_w(b)

    return gather_kernel(pos3, weight)


def kernel(positions, weight):
    flat = positions.reshape(-1)
    out = _gather_call(flat, weight, chunk=16)
    return out.reshape(positions.shape + (weight.shape[1],))


# speedup vs baseline: 2.3848x; 1.0013x over previous
import functools

import jax
import jax.numpy as jnp
from jax import lax
from jax.experimental import pallas as pl
from jax.experimental.pallas import tpu as pltpu
from jax.experimental.pallas import tpu_sc as plsc

_INFO = plsc.get_sparse_core_info()
_NC = _INFO.num_cores
_NS = _INFO.num_subcores
_NW = _NC * _NS

_NB = 3


def _gather_call(positions_flat, weight, chunk):
    b_total = positions_flat.shape[0]
    d = weight.shape[1]
    b_per_w = b_total // _NW
    nchunk = b_per_w // chunk
    nouter = nchunk // _NB
    ntail = nchunk - nouter * _NB
    pos3 = positions_flat.reshape(_NW, nchunk, chunk)
    mesh = plsc.VectorSubcoreMesh(core_axis_name="c", subcore_axis_name="s")

    scratch = (
        [pltpu.VMEM((nchunk, chunk), jnp.int32)]
        + [pltpu.VMEM((chunk, d), jnp.float32) for _ in range(_NB)]
        + [pltpu.SemaphoreType.DMA for _ in range(2 * _NB)]
    )

    @functools.partial(
        pl.kernel,
        mesh=mesh,
        out_type=jax.ShapeDtypeStruct((b_total, d), jnp.float32),
        scratch_types=scratch,
    )
    def gather_kernel(pos_hbm, table_hbm, out_hbm, idx_v, *rest):
        bufs = rest[:_NB]
        gsems = rest[_NB:2 * _NB]
        wsems = rest[2 * _NB:]

        wid = lax.axis_index("s") * _NC + lax.axis_index("c")
        base = wid * b_per_w
        pltpu.sync_copy(pos_hbm.at[wid], idx_v)

        def start_g(c, b):
            pltpu.async_copy(table_hbm.at[idx_v.at[c]], bufs[b], gsems[b])

        def wait_g(c, b):
            pltpu.make_async_copy(table_hbm.at[idx_v.at[c]], bufs[b],
                                  gsems[b]).wait()

        def start_w(c, b):
            pltpu.async_copy(bufs[b], out_hbm.at[pl.ds(base + c * chunk, chunk)],
                             wsems[b])

        def wait_w(b):
            pltpu.make_async_copy(bufs[b], out_hbm.at[pl.ds(base, chunk)],
                                  wsems[b]).wait()

        for b in range(_NB):
            start_g(b, b)

        def slot(c, b, rearm_pred):
            pb = (b - 1) % _NB

            @pl.when(rearm_pred)
            def _():
                wait_w(pb)
                start_g(c - 1 + _NB, pb)

            wait_g(c, b)
            start_w(c, b)

        def body(o, carry):
            for b in range(_NB):
                c = o * _NB + b
                pred = (c >= 1) & (c - 1 + _NB < nchunk)
                slot(c, b, pred)
            return carry

        lax.fori_loop(0, nouter, body, 0)

        for t in range(ntail):
            c = nouter * _NB + t
            slot(c, c % _NB, (c >= 1) & (c - 1 + _NB < nchunk))

        for b in range(_NB):
            wait_w(b)

    return gather_kernel(pos3, weight)


def kernel(positions, weight):
    flat = positions.reshape(-1)
    out = _gather_call(flat, weight, chunk=32)
    return out.reshape(positions.shape + (weight.shape[1],))


# speedup vs baseline: 2.3859x; 1.0005x over previous
import functools

import jax
import jax.numpy as jnp
from jax import lax
from jax.experimental import pallas as pl
from jax.experimental.pallas import tpu as pltpu
from jax.experimental.pallas import tpu_sc as plsc

_INFO = plsc.get_sparse_core_info()
_NC = _INFO.num_cores
_NS = _INFO.num_subcores
_NW = _NC * _NS

_NB = 6


def _gather_call(positions_flat, weight, chunk):
    b_total = positions_flat.shape[0]
    d = weight.shape[1]
    b_per_w = b_total // _NW
    nchunk = b_per_w // chunk
    nouter = nchunk // _NB
    ntail = nchunk - nouter * _NB
    pos3 = positions_flat.reshape(_NW, nchunk, chunk)
    mesh = plsc.VectorSubcoreMesh(core_axis_name="c", subcore_axis_name="s")

    scratch = (
        [pltpu.VMEM((nchunk, chunk), jnp.int32)]
        + [pltpu.VMEM((chunk, d), jnp.float32) for _ in range(_NB)]
        + [pltpu.SemaphoreType.DMA for _ in range(2 * _NB)]
    )

    @functools.partial(
        pl.kernel,
        mesh=mesh,
        out_type=jax.ShapeDtypeStruct((b_total, d), jnp.float32),
        scratch_types=scratch,
    )
    def gather_kernel(pos_hbm, table_hbm, out_hbm, idx_v, *rest):
        bufs = rest[:_NB]
        gsems = rest[_NB:2 * _NB]
        wsems = rest[2 * _NB:]

        wid = lax.axis_index("s") * _NC + lax.axis_index("c")
        base = wid * b_per_w
        pltpu.sync_copy(pos_hbm.at[wid], idx_v)

        def start_g(c, b):
            pltpu.async_copy(table_hbm.at[idx_v.at[c]], bufs[b], gsems[b])

        def wait_g(c, b):
            pltpu.make_async_copy(table_hbm.at[idx_v.at[c]], bufs[b],
                                  gsems[b]).wait()

        def start_w(c, b):
            pltpu.async_copy(bufs[b], out_hbm.at[pl.ds(base + c * chunk, chunk)],
                             wsems[b])

        def wait_w(b):
            pltpu.make_async_copy(bufs[b], out_hbm.at[pl.ds(base, chunk)],
                                  wsems[b]).wait()

        for b in range(_NB):
            start_g(b, b)

        def slot(c, b, rearm_pred):
            pb = (b - 1) % _NB

            @pl.when(rearm_pred)
            def _():
                wait_w(pb)
                start_g(c - 1 + _NB, pb)

            wait_g(c, b)
            start_w(c, b)

        def body(o, carry):
            for b in range(_NB):
                c = o * _NB + b
                pred = (c >= 1) & (c - 1 + _NB < nchunk)
                slot(c, b, pred)
            return carry

        lax.fori_loop(0, nouter, body, 0)

        for t in range(ntail):
            c = nouter * _NB + t
            slot(c, c % _NB, (c >= 1) & (c - 1 + _NB < nchunk))

        for b in range(_NB):
            wait_w(b)

    return gather_kernel(pos3, weight)


def kernel(positions, weight):
    flat = positions.reshape(-1)
    out = _gather_call(flat, weight, chunk=16)
    return out.reshape(positions.shape + (weight.shape[1],))
